# baseline (speedup 1.0000x reference)
import functools
import jax
import jax.numpy as jnp
from jax import lax
from jax.experimental import pallas as pl
from jax.experimental.pallas import tpu as pltpu
from jax.experimental.pallas import tpu_sc as plsc

D = 2048
L = 16
SCALE = float(D) ** 0.5

_info = plsc.get_sparse_core_info()
NC = _info.num_cores
NS = _info.num_subcores
NW = NC * NS

B = 16384
BPW = B // NW
C = 8
NCHUNK = BPW // C
NBUF = 4
DEFER = 1

_mesh = plsc.VectorSubcoreMesh(core_axis_name="c", subcore_axis_name="s")


@functools.partial(
    pl.kernel,
    mesh=_mesh,
    out_type=jax.ShapeDtypeStruct((B, D), jnp.float32),
    scratch_types=[
        pltpu.VMEM((BPW,), jnp.int32),
        pltpu.VMEM((NBUF, C, D), jnp.float32),
        pltpu.SemaphoreType.DMA((NBUF,)),
        pltpu.SemaphoreType.DMA((NBUF,)),
    ],
)
def _embed(idx_hbm, table_hbm, out_hbm, idx_v, rows_v, gsem, ssem):
    wid = lax.axis_index("s") * NC + lax.axis_index("c")
    base = wid * BPW
    pltpu.sync_copy(idx_hbm.at[pl.ds(base, BPW)], idx_v)

    def g_copy(ci, buf):
        return pltpu.make_async_copy(
            table_hbm.at[idx_v.at[pl.ds(ci * C, C)]],
            rows_v.at[buf],
            gsem.at[buf],
        )

    def s_copy(ci, buf):
        return pltpu.make_async_copy(
            rows_v.at[buf],
            out_hbm.at[pl.ds(base + ci * C, C)],
            ssem.at[buf],
        )

    def scale(buf):
        def row_body(r, carry):
            for c in range(D // L):
                sl = pl.ds(c * L, L)
                rows_v[buf, r, sl] = rows_v[buf, r, sl] * SCALE
            return carry
        lax.fori_loop(0, C, row_body, 0)

    for b in range(NBUF):
        g_copy(b, b).start()

    def chunk_body(ci, carry):
        b = lax.rem(ci, NBUF)
        g_copy(ci, b).wait()
        scale(b)
        s_copy(ci, b).start()
        pb = lax.rem(b - DEFER + NBUF, NBUF)
        cj = ci + NBUF - DEFER

        @pl.when((ci >= DEFER) & (cj < NCHUNK))
        def _refill():
            s_copy(cj - NBUF, pb).wait()
            g_copy(cj, pb).start()

        return carry

    lax.fori_loop(0, NCHUNK, chunk_body, 0)

    for ci in range(NCHUNK - NBUF, NCHUNK):
        s_copy(ci, ci % NBUF).wait()


def kernel(input_ids, table):
    ids = input_ids.reshape(-1).astype(jnp.int32)
    out = _embed(ids, table)
    return out.reshape(input_ids.shape + (table.shape[1],))


# speedup vs baseline: 1.0214x; 1.0214x over previous
import functools
import jax
import jax.numpy as jnp
from jax import lax
from jax.experimental import pallas as pl
from jax.experimental.pallas import tpu as pltpu
from jax.experimental.pallas import tpu_sc as plsc

D = 2048
L = 16
SCALE = float(D) ** 0.5

_info = plsc.get_sparse_core_info()
NC = _info.num_cores
NS = _info.num_subcores
NW = NC * NS

B = 16384
BPW = B // NW
C = 8
NCHUNK = BPW // C
NBUF = 7
DEFER = 1

_mesh = plsc.VectorSubcoreMesh(core_axis_name="c", subcore_axis_name="s")


@functools.partial(
    pl.kernel,
    mesh=_mesh,
    out_type=jax.ShapeDtypeStruct((B, D), jnp.float32),
    scratch_types=[
        pltpu.VMEM((BPW,), jnp.int32),
        pltpu.VMEM((NBUF, C, D), jnp.float32),
        pltpu.SemaphoreType.DMA((NBUF,)),
        pltpu.SemaphoreType.DMA((NBUF,)),
    ],
)
def _embed(idx_hbm, table_hbm, out_hbm, idx_v, rows_v, gsem, ssem):
    wid = lax.axis_index("s") * NC + lax.axis_index("c")
    base = wid * BPW
    pltpu.sync_copy(idx_hbm.at[pl.ds(base, BPW)], idx_v)

    def g_copy(ci, buf):
        return pltpu.make_async_copy(
            table_hbm.at[idx_v.at[pl.ds(ci * C, C)]],
            rows_v.at[buf],
            gsem.at[buf],
        )

    def s_copy(ci, buf):
        return pltpu.make_async_copy(
            rows_v.at[buf],
            out_hbm.at[pl.ds(base + ci * C, C)],
            ssem.at[buf],
        )

    def scale(buf):
        def row_body(r, carry):
            for c in range(D // L):
                sl = pl.ds(c * L, L)
                rows_v[buf, r, sl] = rows_v[buf, r, sl] * SCALE
            return carry
        lax.fori_loop(0, C, row_body, 0)

    for b in range(NBUF):
        g_copy(b, b).start()

    def chunk_body(ci, carry):
        b = lax.rem(ci, NBUF)
        g_copy(ci, b).wait()
        scale(b)
        s_copy(ci, b).start()
        pb = lax.rem(b - DEFER + NBUF, NBUF)
        cj = ci + NBUF - DEFER

        @pl.when((ci >= DEFER) & (cj < NCHUNK))
        def _refill():
            s_copy(cj - NBUF, pb).wait()
            g_copy(cj, pb).start()

        return carry

    lax.fori_loop(0, NCHUNK, chunk_body, 0)

    for ci in range(NCHUNK - NBUF, NCHUNK):
        s_copy(ci, ci % NBUF).wait()


def kernel(input_ids, table):
    ids = input_ids.reshape(-1).astype(jnp.int32)
    out = _embed(ids, table)
    return out.reshape(input_ids.shape + (table.shape[1],))


# speedup vs baseline: 1.0233x; 1.0018x over previous
import functools
import jax
import jax.numpy as jnp
from jax import lax
from jax.experimental import pallas as pl
from jax.experimental.pallas import tpu as pltpu
from jax.experimental.pallas import tpu_sc as plsc

D = 2048
L = 16
SCALE = float(D) ** 0.5

_info = plsc.get_sparse_core_info()
NC = _info.num_cores
NS = _info.num_subcores
NW = NC * NS

B = 16384
BPW = B // NW
C = 8
NCHUNK = BPW // C
NBUF = 7
DEFER = 2

_mesh = plsc.VectorSubcoreMesh(core_axis_name="c", subcore_axis_name="s")


@functools.partial(
    pl.kernel,
    mesh=_mesh,
    out_type=jax.ShapeDtypeStruct((B, D), jnp.float32),
    scratch_types=[
        pltpu.VMEM((BPW,), jnp.int32),
        pltpu.VMEM((NBUF, C, D), jnp.float32),
        pltpu.SemaphoreType.DMA((NBUF,)),
        pltpu.SemaphoreType.DMA((NBUF,)),
    ],
)
def _embed(idx_hbm, table_hbm, out_hbm, idx_v, rows_v, gsem, ssem):
    wid = lax.axis_index("s") * NC + lax.axis_index("c")
    base = wid * BPW
    pltpu.sync_copy(idx_hbm.at[pl.ds(base, BPW)], idx_v)

    def g_copy(ci, buf):
        return pltpu.make_async_copy(
            table_hbm.at[idx_v.at[pl.ds(ci * C, C)]],
            rows_v.at[buf],
            gsem.at[buf],
        )

    def s_copy(ci, buf):
        return pltpu.make_async_copy(
            rows_v.at[buf],
            out_hbm.at[pl.ds(base + ci * C, C)],
            ssem.at[buf],
        )

    def scale(buf):
        def row_body(r, carry):
            for c in range(D // L):
                sl = pl.ds(c * L, L)
                rows_v[buf, r, sl] = rows_v[buf, r, sl] * SCALE
            return carry
        lax.fori_loop(0, C, row_body, 0)

    for b in range(NBUF):
        g_copy(b, b).start()

    def chunk_body(ci, carry):
        b = lax.rem(ci, NBUF)
        g_copy(ci, b).wait()
        scale(b)
        s_copy(ci, b).start()
        pb = lax.rem(b - DEFER + NBUF, NBUF)
        cj = ci + NBUF - DEFER

        @pl.when((ci >= DEFER) & (cj < NCHUNK))
        def _refill():
            s_copy(cj - NBUF, pb).wait()
            g_copy(cj, pb).start()

        return carry

    lax.fori_loop(0, NCHUNK, chunk_body, 0)

    for ci in range(NCHUNK - NBUF, NCHUNK):
        s_copy(ci, ci % NBUF).wait()


def kernel(input_ids, table):
    ids = input_ids.reshape(-1).astype(jnp.int32)
    out = _embed(ids, table)
    return out.reshape(input_ids.shape + (table.shape[1],))
